# baseline (speedup 1.0000x reference)
import functools

import jax
import jax.numpy as jnp
from jax import lax
from jax.experimental import pallas as pl
from jax.experimental.pallas import tpu as pltpu
from jax.experimental.pallas import tpu_sc as plsc

K_TOP_CONST = 100

QB = 256
CB = 1024
C_REAL = 100000
C_PAD = 100352
CHUNK = 128
NCH = C_PAD // CHUNK
NCHP = 896
NV_M = NCHP // 16
NCC = 2
NSC = 16
NW = NCC * NSC
QPW = 1024 // NW
GCAP = 224
GHALF = GCAP // 2
NV_SG = 16
SVCAP = 1024
OUTW = 104

NEG = float("-inf")


def _mm_body(q_ref, c_ref, s_ref, mt_ref):
    j = pl.program_id(1)
    nj = pl.num_programs(1)
    del nj
    s = lax.dot_general(q_ref[...], c_ref[...], (((1,), (1,)), ((), ())),
                        preferred_element_type=jnp.float32)
    s = jnp.where(
        lax.broadcasted_iota(jnp.int32, (QB, CB), 1) + j * CB >= C_REAL,
        NEG, s)
    s3 = s.reshape(QB, CB // CHUNK, CHUNK)
    s_ref[...] = s3
    cm = jnp.max(s3, axis=2)
    mt_ref[...] = cm.T


def _stage1(queries, cand_pad):
    Q, D = queries.shape
    return pl.pallas_call(
        _mm_body,
        grid=(Q // QB, C_PAD // CB),
        in_specs=[
            pl.BlockSpec((QB, D), lambda i, j: (i, 0)),
            pl.BlockSpec((CB, D), lambda i, j: (j, 0)),
        ],
        out_specs=[
            pl.BlockSpec((QB, CB // CHUNK, CHUNK), lambda i, j: (i, j, 0)),
            pl.BlockSpec((CB // CHUNK, QB), lambda i, j: (j, i)),
        ],
        out_shape=[
            jax.ShapeDtypeStruct((Q, NCH, CHUNK), jnp.float32),
            jax.ShapeDtypeStruct((NCH, Q), jnp.float32),
        ],
    )(queries, cand_pad)


def _sort_kv(k, v):
    return plsc.sort_key_val(k, v, descending=True)


def _vsort_desc(x):
    k, _ = _sort_kv(x, x)
    return k


def _rev_run(b):
    return [lax.rev(x, (0,)) for x in reversed(b)]


def _bitonic_finish_k(v):
    n = len(v)
    d = n // 2
    while d >= 1:
        nv = list(v)
        for base in range(0, n, 2 * d):
            for i in range(base, base + d):
                nv[i] = jnp.maximum(v[i], v[i + d])
                nv[i + d] = jnp.minimum(v[i], v[i + d])
        v = nv
        d //= 2
    return [_vsort_desc(x) for x in v]


def _merge_desc_k(a, b):
    return _bitonic_finish_k(a + _rev_run(b))


def _merge_desc_k_top(a, b):
    m = len(a)
    b2 = _rev_run(b)
    v = [jnp.maximum(a[i], b2[i]) for i in range(m)]
    if m == 1:
        return [_vsort_desc(v[0])]
    return _bitonic_finish_k(v)


def _bitonic_finish_kv(ks, vs):
    n = len(ks)
    d = n // 2
    while d >= 1:
        nk, nv = list(ks), list(vs)
        for base in range(0, n, 2 * d):
            for i in range(base, base + d):
                c = ks[i] >= ks[i + d]
                nk[i] = jnp.where(c, ks[i], ks[i + d])
                nv[i] = jnp.where(c, vs[i], vs[i + d])
                nk[i + d] = jnp.where(c, ks[i + d], ks[i])
                nv[i + d] = jnp.where(c, vs[i + d], vs[i])
        ks, vs = nk, nv
        d //= 2
    out = [_sort_kv(k, v) for k, v in zip(ks, vs)]
    return [k for k, _ in out], [v for _, v in out]


def _merge_desc_kv(ka, va, kb, vb):
    return _bitonic_finish_kv(ka + _rev_run(kb), va + _rev_run(vb))


def _merge_desc_kv_top(ka, va, kb, vb):
    m = len(ka)
    kb2, vb2 = _rev_run(kb), _rev_run(vb)
    ks, vs = [], []
    for i in range(m):
        c = ka[i] >= kb2[i]
        ks.append(jnp.where(c, ka[i], kb2[i]))
        vs.append(jnp.where(c, va[i], vb2[i]))
    if m == 1:
        k, v = _sort_kv(ks[0], vs[0])
        return [k], [v]
    return _bitonic_finish_kv(ks, vs)


def _tie_fixup(ov_v, oi_v, phases=4):
    iota = lax.iota(jnp.int32, 16)
    for p in range(phases):
        parity = p % 2
        new_v = []
        for i in range(8):
            pos = 16 * i + iota
            step = jnp.where(pos % 2 == parity, 1, -1)
            partner = jnp.clip(pos + step, 0, 127)
            k = ov_v[pl.ds(16 * i, 16)]
            v = oi_v[pl.ds(16 * i, 16)]
            kp = plsc.load_gather(ov_v, [partner])
            vp = plsc.load_gather(oi_v, [partner])
            eq = k == kp
            lead = partner > pos
            nv = jnp.where(lead, jnp.minimum(v, vp), jnp.maximum(v, vp))
            new_v.append(jnp.where(eq, nv, v))
        for i in range(8):
            oi_v[pl.ds(16 * i, 16)] = new_v[i]


def _topk_ladder_k(vregs, keep):
    runs = [[_vsort_desc(x)] for x in vregs]
    while len(runs) > 1:
        nxt = []
        for i in range(0, len(runs) - 1, 2):
            a, b = runs[i], runs[i + 1]
            if len(a) < keep:
                nxt.append(_merge_desc_k(a, b))
            else:
                nxt.append(_merge_desc_k_top(a, b))
        if len(runs) % 2:
            nxt.append(runs[-1])
        runs = nxt
    return runs[0]


def _sort8_kv(ks, vs):
    runs = []
    for k, v in zip(ks, vs):
        k2, v2 = _sort_kv(k, v)
        runs.append(([k2], [v2]))
    while len(runs) > 1:
        nxt = []
        for i in range(0, len(runs), 2):
            ka, va = runs[i]
            kb, vb = runs[i + 1]
            nxt.append(_merge_desc_kv(ka, va, kb, vb))
        runs = nxt
    return runs[0]


def _sc_body(m_hbm, rows_hbm, outv_hbm, outi_hbm,
             m_v0, m_v1, tau_v, gid_v0, gid_v1, ga0, gb0, ga1, gb1,
             rows_v0, rows_v1, sv_s, sv_i, ov_v, oi_v,
             semm0, semm1, semg0, semg1):
    wid = lax.axis_index("s") * NCC + lax.axis_index("c")
    base = wid * QPW
    iota = lax.iota(jnp.int32, 16)

    def start_mload(q, m_v, semm):
        qq = jnp.minimum(q, 1024 - 1)
        pltpu.async_copy(m_hbm.at[pl.ds(qq * NCHP, NCHP)], m_v, semm)

    def wait_mload(m_v, semm):
        pltpu.make_async_copy(m_hbm.at[pl.ds(0, NCHP)], m_v, semm).wait()

    def start_gather(ga, gb, rows_v, semg):
        pltpu.async_copy(rows_hbm.at[ga], rows_v.at[pl.ds(0, GHALF)], semg)
        pltpu.async_copy(rows_hbm.at[gb], rows_v.at[pl.ds(GHALF, GHALF)],
                         semg)

    def wait_gather(ga, gb, rows_v, semg):
        pltpu.make_async_copy(rows_hbm.at[ga],
                              rows_v.at[pl.ds(0, GHALF)], semg).wait()
        pltpu.make_async_copy(rows_hbm.at[gb],
                              rows_v.at[pl.ds(GHALF, GHALF)], semg).wait()

    def compute_phase(q, m_v, gid_v, ga, gb):
        mv = [m_v[pl.ds(16 * i, 16)] for i in range(NV_M)]
        sg = [jnp.maximum(jnp.maximum(mv[4 * i], mv[4 * i + 1]),
                          jnp.maximum(mv[4 * i + 2], mv[4 * i + 3]))
              for i in range(NV_M // 4)]
        negv = jnp.full((16,), NEG, jnp.float32)
        sg += [negv] * (NV_SG - len(sg))
        run = _topk_ladder_k(sg, 8)
        tau_v[...] = run[6]
        tau = plsc.load_gather(tau_v, [jnp.full((16,), 3, jnp.int32)])

        for i in range(GCAP // 16):
            gid_v[pl.ds(16 * i, 16)] = jnp.zeros((16,), jnp.int32)

        def comp_body(i, off):
            mk = m_v[pl.ds(16 * i, 16)]
            msk = mk >= tau
            rowid = q * NCH + 16 * i + iota
            plsc.store_compressed(gid_v.at[pl.ds(off, 16)], rowid, mask=msk)
            cnt = jnp.sum(msk.astype(jnp.int32))
            return jnp.minimum(off + cnt, GCAP - 16)

        nch = lax.fori_loop(0, NV_M, comp_body, jnp.int32(0), unroll=True)
        for i in range(GHALF // 16):
            ga[pl.ds(16 * i, 16)] = gid_v[pl.ds(16 * i, 16)]
            gb[pl.ds(16 * i, 16)] = gid_v[pl.ds(GHALF + 16 * i, 16)]
        return tau, nch

    def finish_phase(q, tau, nch, gid_v, rows_v):
        def filt_body(g, off):
            rid = plsc.load_gather(gid_v, [jnp.full((16,), 0, jnp.int32) + g])
            cbase = (rid - q * NCH) * CHUNK
            for ti in range(CHUNK // 16):
                s = rows_v[g, pl.ds(16 * ti, 16)]
                msk = s >= tau
                plsc.store_compressed(sv_s.at[pl.ds(off, 16)], s, mask=msk)
                idxv = cbase + 16 * ti + iota
                plsc.store_compressed(sv_i.at[pl.ds(off, 16)], idxv, mask=msk)
                off = jnp.minimum(off + jnp.sum(msk.astype(jnp.int32)),
                                  SVCAP - 16)
            return off

        svcnt = lax.fori_loop(0, nch, filt_body, jnp.int32(0))

        def load_group(g):
            ks, vs = [], []
            for ti in range(8):
                pos = 128 * g + 16 * ti + iota
                k = sv_s[pl.ds(128 * g + 16 * ti, 16)]
                ks.append(jnp.where(pos < svcnt, k, NEG))
                vs.append(sv_i[pl.ds(128 * g + 16 * ti, 16)])
            return ks, vs

        ks0, vs0 = load_group(jnp.int32(0))
        rk, rv = _sort8_kv(ks0, vs0)

        def grp_body(g, carry):
            rk = list(carry[:8])
            rv = list(carry[8:])
            ks, vs = load_group(g)
            gk, gv = _sort8_kv(ks, vs)
            rk, rv = _merge_desc_kv_top(rk, rv, gk, gv)
            return tuple(rk) + tuple(rv)

        ngrp = (jnp.minimum(svcnt, SVCAP) + 127) // 128
        carry = lax.fori_loop(1, ngrp, grp_body, tuple(rk) + tuple(rv))
        rk, rv = list(carry[:8]), list(carry[8:])

        for ti in range(8):
            ov_v[pl.ds(16 * ti, 16)] = rk[ti]
            oi_v[pl.ds(16 * ti, 16)] = rv[ti]
        _tie_fixup(ov_v, oi_v)
        pltpu.sync_copy(ov_v.at[pl.ds(0, OUTW)],
                        outv_hbm.at[pl.ds(q * OUTW, OUTW)])
        pltpu.sync_copy(oi_v.at[pl.ds(0, OUTW)],
                        outi_hbm.at[pl.ds(q * OUTW, OUTW)])

    start_mload(base + 0, m_v0, semm0)
    start_mload(base + 1, m_v1, semm1)

    wait_mload(m_v0, semm0)
    tau_e, nch_e = compute_phase(base + 0, m_v0, gid_v0, ga0, gb0)
    start_gather(ga0, gb0, rows_v0, semg0)
    start_mload(base + 2, m_v0, semm0)

    wait_mload(m_v1, semm1)
    tau_o, nch_o = compute_phase(base + 1, m_v1, gid_v1, ga1, gb1)
    start_gather(ga1, gb1, rows_v1, semg1)
    start_mload(base + 3, m_v1, semm1)

    wait_gather(ga0, gb0, rows_v0, semg0)
    finish_phase(base + 0, tau_e, nch_e, gid_v0, rows_v0)

    def pair_body(p, carry):
        tau_o, nch_o = carry
        a = base + 2 * p
        b = a + 1

        wait_mload(m_v0, semm0)
        tau_e, nch_e = compute_phase(a, m_v0, gid_v0, ga0, gb0)
        start_gather(ga0, gb0, rows_v0, semg0)
        start_mload(a + 2, m_v0, semm0)

        wait_gather(ga1, gb1, rows_v1, semg1)
        finish_phase(b - 2, tau_o, nch_o, gid_v1, rows_v1)

        wait_mload(m_v1, semm1)
        tau_o, nch_o = compute_phase(b, m_v1, gid_v1, ga1, gb1)
        start_gather(ga1, gb1, rows_v1, semg1)
        start_mload(b + 2, m_v1, semm1)

        wait_gather(ga0, gb0, rows_v0, semg0)
        finish_phase(a, tau_e, nch_e, gid_v0, rows_v0)
        return (tau_o, nch_o)

    tau_o, nch_o = lax.fori_loop(1, QPW // 2, pair_body, (tau_o, nch_o))

    wait_gather(ga1, gb1, rows_v1, semg1)
    finish_phase(base + QPW - 1, tau_o, nch_o, gid_v1, rows_v1)


def _stage2(m_flat, rows, Q):
    mesh = plsc.VectorSubcoreMesh(core_axis_name="c", subcore_axis_name="s",
                                  num_cores=NCC, num_subcores=NSC)
    f = functools.partial(
        pl.kernel,
        out_type=[
            jax.ShapeDtypeStruct((Q * OUTW,), jnp.float32),
            jax.ShapeDtypeStruct((Q * OUTW,), jnp.int32),
        ],
        mesh=mesh,
        compiler_params=pltpu.CompilerParams(needs_layout_passes=False),
        scratch_types=[
            pltpu.VMEM((NCHP,), jnp.float32),
            pltpu.VMEM((NCHP,), jnp.float32),
            pltpu.VMEM((16,), jnp.float32),
            pltpu.VMEM((GCAP,), jnp.int32),
            pltpu.VMEM((GCAP,), jnp.int32),
            pltpu.VMEM((GHALF,), jnp.int32),
            pltpu.VMEM((GHALF,), jnp.int32),
            pltpu.VMEM((GHALF,), jnp.int32),
            pltpu.VMEM((GHALF,), jnp.int32),
            pltpu.VMEM((GCAP, CHUNK), jnp.float32),
            pltpu.VMEM((GCAP, CHUNK), jnp.float32),
            pltpu.VMEM((SVCAP,), jnp.float32),
            pltpu.VMEM((SVCAP,), jnp.int32),
            pltpu.VMEM((128,), jnp.float32),
            pltpu.VMEM((128,), jnp.int32),
            pltpu.SemaphoreType.DMA,
            pltpu.SemaphoreType.DMA,
            pltpu.SemaphoreType.DMA,
            pltpu.SemaphoreType.DMA,
        ],
    )(_sc_body)
    return f(m_flat, rows)


def kernel(queries, candidates):
    Q, D = queries.shape
    C = candidates.shape[0]
    cand_pad = jnp.pad(candidates, ((0, C_PAD - C), (0, 0)))
    scores, mt = _stage1(queries, cand_pad)
    m = jnp.pad(mt.T, ((0, 0), (0, NCHP - NCH)), constant_values=NEG)
    rows = scores.reshape(Q * NCH, CHUNK)
    outv, outi = _stage2(m.reshape(-1), rows, Q)
    top_scores = outv.reshape(Q, OUTW)[:, :K_TOP_CONST]
    top_idx = outi.reshape(Q, OUTW)[:, :K_TOP_CONST]
    return (top_scores, top_idx)


# speedup vs baseline: 4.7389x; 4.7389x over previous
import functools

import jax
import jax.numpy as jnp
from jax import lax
from jax.experimental import pallas as pl
from jax.experimental.pallas import tpu as pltpu
from jax.experimental.pallas import tpu_sc as plsc

K_TOP_CONST = 100

QB = 256
CB = 1024
C_REAL = 100000
C_PAD = 100352
CHUNK = 128
NCH = C_PAD // CHUNK
NCHP = 896
NV_M = NCHP // 16
NCC = 2
NSC = 16
NW = NCC * NSC
QPW = 1024 // NW
GCAP = 224
GHALF = GCAP // 2
NV_SG = 16
SVCAP = 1024
OUTW = 104

NEG = float("-inf")


def _mm_body(q_ref, c_ref, s_ref, mt_ref):
    j = pl.program_id(1)
    nj = pl.num_programs(1)
    del nj
    s = lax.dot_general(q_ref[...], c_ref[...], (((1,), (1,)), ((), ())),
                        preferred_element_type=jnp.float32)
    s = jnp.where(
        lax.broadcasted_iota(jnp.int32, (QB, CB), 1) + j * CB >= C_REAL,
        NEG, s)
    s3 = s.reshape(QB, CB // CHUNK, CHUNK)
    s_ref[...] = s3
    cm = jnp.max(s3, axis=2)
    mt_ref[...] = cm.T


def _stage1(queries, cand_pad):
    Q, D = queries.shape
    return pl.pallas_call(
        _mm_body,
        grid=(Q // QB, C_PAD // CB),
        in_specs=[
            pl.BlockSpec((QB, D), lambda i, j: (i, 0)),
            pl.BlockSpec((CB, D), lambda i, j: (j, 0)),
        ],
        out_specs=[
            pl.BlockSpec((QB, CB // CHUNK, CHUNK), lambda i, j: (i, j, 0)),
            pl.BlockSpec((CB // CHUNK, QB), lambda i, j: (j, i)),
        ],
        out_shape=[
            jax.ShapeDtypeStruct((Q, NCH, CHUNK), jnp.float32),
            jax.ShapeDtypeStruct((NCH, Q), jnp.float32),
        ],
    )(queries, cand_pad)


def _sort_kv(k, v):
    return plsc.sort_key_val(k, v, descending=True)


def _vsort_desc(x):
    k, _ = _sort_kv(x, x)
    return k


def _rev_run(b):
    return [lax.rev(x, (0,)) for x in reversed(b)]


def _bitonic_finish_k(v):
    n = len(v)
    d = n // 2
    while d >= 1:
        nv = list(v)
        for base in range(0, n, 2 * d):
            for i in range(base, base + d):
                nv[i] = jnp.maximum(v[i], v[i + d])
                nv[i + d] = jnp.minimum(v[i], v[i + d])
        v = nv
        d //= 2
    return [_vsort_desc(x) for x in v]


def _merge_desc_k(a, b):
    return _bitonic_finish_k(a + _rev_run(b))


def _merge_desc_k_top(a, b):
    m = len(a)
    b2 = _rev_run(b)
    v = [jnp.maximum(a[i], b2[i]) for i in range(m)]
    if m == 1:
        return [_vsort_desc(v[0])]
    return _bitonic_finish_k(v)


def _bitonic_finish_kv(ks, vs):
    n = len(ks)
    d = n // 2
    while d >= 1:
        nk, nv = list(ks), list(vs)
        for base in range(0, n, 2 * d):
            for i in range(base, base + d):
                c = ks[i] >= ks[i + d]
                nk[i] = jnp.where(c, ks[i], ks[i + d])
                nv[i] = jnp.where(c, vs[i], vs[i + d])
                nk[i + d] = jnp.where(c, ks[i + d], ks[i])
                nv[i + d] = jnp.where(c, vs[i + d], vs[i])
        ks, vs = nk, nv
        d //= 2
    out = [_sort_kv(k, v) for k, v in zip(ks, vs)]
    return [k for k, _ in out], [v for _, v in out]


def _merge_desc_kv(ka, va, kb, vb):
    return _bitonic_finish_kv(ka + _rev_run(kb), va + _rev_run(vb))


def _merge_desc_kv_top(ka, va, kb, vb):
    m = len(ka)
    kb2, vb2 = _rev_run(kb), _rev_run(vb)
    ks, vs = [], []
    for i in range(m):
        c = ka[i] >= kb2[i]
        ks.append(jnp.where(c, ka[i], kb2[i]))
        vs.append(jnp.where(c, va[i], vb2[i]))
    if m == 1:
        k, v = _sort_kv(ks[0], vs[0])
        return [k], [v]
    return _bitonic_finish_kv(ks, vs)


def _tie_fixup(ov_v, oi_v, phases=4):
    iota = lax.iota(jnp.int32, 16)
    for p in range(phases):
        parity = p % 2
        new_v = []
        for i in range(8):
            pos = 16 * i + iota
            step = jnp.where(pos % 2 == parity, 1, -1)
            partner = jnp.clip(pos + step, 0, 127)
            k = ov_v[pl.ds(16 * i, 16)]
            v = oi_v[pl.ds(16 * i, 16)]
            kp = plsc.load_gather(ov_v, [partner])
            vp = plsc.load_gather(oi_v, [partner])
            eq = k == kp
            lead = partner > pos
            nv = jnp.where(lead, jnp.minimum(v, vp), jnp.maximum(v, vp))
            new_v.append(jnp.where(eq, nv, v))
        for i in range(8):
            oi_v[pl.ds(16 * i, 16)] = new_v[i]


def _topk_ladder_k(vregs, keep):
    runs = [[_vsort_desc(x)] for x in vregs]
    while len(runs) > 1:
        nxt = []
        for i in range(0, len(runs) - 1, 2):
            a, b = runs[i], runs[i + 1]
            if len(a) < keep:
                nxt.append(_merge_desc_k(a, b))
            else:
                nxt.append(_merge_desc_k_top(a, b))
        if len(runs) % 2:
            nxt.append(runs[-1])
        runs = nxt
    return runs[0]


def _sort8_kv(ks, vs):
    runs = []
    for k, v in zip(ks, vs):
        k2, v2 = _sort_kv(k, v)
        runs.append(([k2], [v2]))
    while len(runs) > 1:
        nxt = []
        for i in range(0, len(runs), 2):
            ka, va = runs[i]
            kb, vb = runs[i + 1]
            nxt.append(_merge_desc_kv(ka, va, kb, vb))
        runs = nxt
    return runs[0]


def _sc_body(m_hbm, rows_hbm, outv_hbm, outi_hbm,
             m_v0, m_v1, tau_v, gid_v0, gid_v1, ga0, gb0, ga1, gb1,
             rows_v0, rows_v1, sv_s, sv_i, ov_v, oi_v,
             semm0, semm1, semg0, semg1):
    wid = lax.axis_index("s") * NCC + lax.axis_index("c")
    base = wid * QPW
    iota = lax.iota(jnp.int32, 16)

    def start_mload(q, m_v, semm):
        qq = jnp.minimum(q, 1024 - 1)
        pltpu.async_copy(m_hbm.at[pl.ds(qq * NCHP, NCHP)], m_v, semm)

    def wait_mload(m_v, semm):
        pltpu.make_async_copy(m_hbm.at[pl.ds(0, NCHP)], m_v, semm).wait()

    def start_gather(ga, gb, rows_v, semg):
        pltpu.async_copy(rows_hbm.at[ga], rows_v.at[pl.ds(0, GHALF)], semg)
        pltpu.async_copy(rows_hbm.at[gb], rows_v.at[pl.ds(GHALF, GHALF)],
                         semg)

    def wait_gather(ga, gb, rows_v, semg):
        pltpu.make_async_copy(rows_hbm.at[ga],
                              rows_v.at[pl.ds(0, GHALF)], semg).wait()
        pltpu.make_async_copy(rows_hbm.at[gb],
                              rows_v.at[pl.ds(GHALF, GHALF)], semg).wait()

    def compute_phase(q, m_v, gid_v, ga, gb):
        mv = [m_v[pl.ds(16 * i, 16)] for i in range(NV_M)]
        sg = [jnp.maximum(jnp.maximum(mv[4 * i], mv[4 * i + 1]),
                          jnp.maximum(mv[4 * i + 2], mv[4 * i + 3]))
              for i in range(NV_M // 4)]
        negv = jnp.full((16,), NEG, jnp.float32)
        sg += [negv] * (NV_SG - len(sg))
        run = _topk_ladder_k(sg, 8)
        tau_v[...] = run[6]
        tau = plsc.load_gather(tau_v, [jnp.full((16,), 3, jnp.int32)])

        for i in range(GCAP // 16):
            gid_v[pl.ds(16 * i, 16)] = q * NCH + 16 * i + iota

        def comp_body(i, off):
            mk = m_v[pl.ds(16 * i, 16)]
            msk = mk >= tau
            rowid = q * NCH + 16 * i + iota
            plsc.store_compressed(gid_v.at[pl.ds(off, 16)], rowid, mask=msk)
            cnt = jnp.sum(msk.astype(jnp.int32))
            return jnp.minimum(off + cnt, GCAP - 16)

        nch = lax.fori_loop(0, NV_M, comp_body, jnp.int32(0), unroll=True)
        for i in range(GHALF // 16):
            ga[pl.ds(16 * i, 16)] = gid_v[pl.ds(16 * i, 16)]
            gb[pl.ds(16 * i, 16)] = gid_v[pl.ds(GHALF + 16 * i, 16)]
        return tau, nch

    def finish_phase(q, tau, nch, gid_v, rows_v):
        def filt_body(g, off):
            rid = plsc.load_gather(gid_v, [jnp.full((16,), 0, jnp.int32) + g])
            cbase = (rid - q * NCH) * CHUNK
            for ti in range(CHUNK // 16):
                s = rows_v[g, pl.ds(16 * ti, 16)]
                msk = s >= tau
                plsc.store_compressed(sv_s.at[pl.ds(off, 16)], s, mask=msk)
                idxv = cbase + 16 * ti + iota
                plsc.store_compressed(sv_i.at[pl.ds(off, 16)], idxv, mask=msk)
                off = jnp.minimum(off + jnp.sum(msk.astype(jnp.int32)),
                                  SVCAP - 16)
            return off

        svcnt = lax.fori_loop(0, nch, filt_body, jnp.int32(0))

        def load_group(g):
            ks, vs = [], []
            for ti in range(8):
                pos = 128 * g + 16 * ti + iota
                k = sv_s[pl.ds(128 * g + 16 * ti, 16)]
                ks.append(jnp.where(pos < svcnt, k, NEG))
                vs.append(sv_i[pl.ds(128 * g + 16 * ti, 16)])
            return ks, vs

        ks0, vs0 = load_group(jnp.int32(0))
        rk, rv = _sort8_kv(ks0, vs0)

        def grp_body(g, carry):
            rk = list(carry[:8])
            rv = list(carry[8:])
            ks, vs = load_group(g)
            gk, gv = _sort8_kv(ks, vs)
            rk, rv = _merge_desc_kv_top(rk, rv, gk, gv)
            return tuple(rk) + tuple(rv)

        ngrp = (jnp.minimum(svcnt, SVCAP) + 127) // 128
        carry = lax.fori_loop(1, ngrp, grp_body, tuple(rk) + tuple(rv))
        rk, rv = list(carry[:8]), list(carry[8:])

        for ti in range(8):
            ov_v[pl.ds(16 * ti, 16)] = rk[ti]
            oi_v[pl.ds(16 * ti, 16)] = rv[ti]
        _tie_fixup(ov_v, oi_v)
        pltpu.sync_copy(ov_v.at[pl.ds(0, OUTW)],
                        outv_hbm.at[pl.ds(q * OUTW, OUTW)])
        pltpu.sync_copy(oi_v.at[pl.ds(0, OUTW)],
                        outi_hbm.at[pl.ds(q * OUTW, OUTW)])

    start_mload(base + 0, m_v0, semm0)
    start_mload(base + 1, m_v1, semm1)

    wait_mload(m_v0, semm0)
    tau_e, nch_e = compute_phase(base + 0, m_v0, gid_v0, ga0, gb0)
    start_gather(ga0, gb0, rows_v0, semg0)
    start_mload(base + 2, m_v0, semm0)

    wait_mload(m_v1, semm1)
    tau_o, nch_o = compute_phase(base + 1, m_v1, gid_v1, ga1, gb1)
    start_gather(ga1, gb1, rows_v1, semg1)
    start_mload(base + 3, m_v1, semm1)

    wait_gather(ga0, gb0, rows_v0, semg0)
    finish_phase(base + 0, tau_e, nch_e, gid_v0, rows_v0)

    def pair_body(p, carry):
        tau_o, nch_o = carry
        a = base + 2 * p
        b = a + 1

        wait_mload(m_v0, semm0)
        tau_e, nch_e = compute_phase(a, m_v0, gid_v0, ga0, gb0)
        start_gather(ga0, gb0, rows_v0, semg0)
        start_mload(a + 2, m_v0, semm0)

        wait_gather(ga1, gb1, rows_v1, semg1)
        finish_phase(b - 2, tau_o, nch_o, gid_v1, rows_v1)

        wait_mload(m_v1, semm1)
        tau_o, nch_o = compute_phase(b, m_v1, gid_v1, ga1, gb1)
        start_gather(ga1, gb1, rows_v1, semg1)
        start_mload(b + 2, m_v1, semm1)

        wait_gather(ga0, gb0, rows_v0, semg0)
        finish_phase(a, tau_e, nch_e, gid_v0, rows_v0)
        return (tau_o, nch_o)

    tau_o, nch_o = lax.fori_loop(1, QPW // 2, pair_body, (tau_o, nch_o))

    wait_gather(ga1, gb1, rows_v1, semg1)
    finish_phase(base + QPW - 1, tau_o, nch_o, gid_v1, rows_v1)


def _stage2(m_flat, rows, Q):
    mesh = plsc.VectorSubcoreMesh(core_axis_name="c", subcore_axis_name="s",
                                  num_cores=NCC, num_subcores=NSC)
    f = functools.partial(
        pl.kernel,
        out_type=[
            jax.ShapeDtypeStruct((Q * OUTW,), jnp.float32),
            jax.ShapeDtypeStruct((Q * OUTW,), jnp.int32),
        ],
        mesh=mesh,
        compiler_params=pltpu.CompilerParams(needs_layout_passes=False),
        scratch_types=[
            pltpu.VMEM((NCHP,), jnp.float32),
            pltpu.VMEM((NCHP,), jnp.float32),
            pltpu.VMEM((16,), jnp.float32),
            pltpu.VMEM((GCAP,), jnp.int32),
            pltpu.VMEM((GCAP,), jnp.int32),
            pltpu.VMEM((GHALF,), jnp.int32),
            pltpu.VMEM((GHALF,), jnp.int32),
            pltpu.VMEM((GHALF,), jnp.int32),
            pltpu.VMEM((GHALF,), jnp.int32),
            pltpu.VMEM((GCAP, CHUNK), jnp.float32),
            pltpu.VMEM((GCAP, CHUNK), jnp.float32),
            pltpu.VMEM((SVCAP,), jnp.float32),
            pltpu.VMEM((SVCAP,), jnp.int32),
            pltpu.VMEM((128,), jnp.float32),
            pltpu.VMEM((128,), jnp.int32),
            pltpu.SemaphoreType.DMA,
            pltpu.SemaphoreType.DMA,
            pltpu.SemaphoreType.DMA,
            pltpu.SemaphoreType.DMA,
        ],
    )(_sc_body)
    return f(m_flat, rows)


def kernel(queries, candidates):
    Q, D = queries.shape
    C = candidates.shape[0]
    cand_pad = jnp.pad(candidates, ((0, C_PAD - C), (0, 0)))
    scores, mt = _stage1(queries, cand_pad)
    m = jnp.pad(mt.T, ((0, 0), (0, NCHP - NCH)), constant_values=NEG)
    rows = scores.reshape(Q * NCH, CHUNK)
    outv, outi = _stage2(m.reshape(-1), rows, Q)
    top_scores = outv.reshape(Q, OUTW)[:, :K_TOP_CONST]
    top_idx = outi.reshape(Q, OUTW)[:, :K_TOP_CONST]
    return (top_scores, top_idx)


# speedup vs baseline: 6.6485x; 1.4030x over previous
import functools

import jax
import jax.numpy as jnp
from jax import lax
from jax.experimental import pallas as pl
from jax.experimental.pallas import tpu as pltpu
from jax.experimental.pallas import tpu_sc as plsc

K_TOP_CONST = 100

QB = 256
CB = 1024
C_REAL = 100000
C_PAD = 100352
CHUNK = 128
NCH = C_PAD // CHUNK
NCHP = 896
NV_M = NCHP // 16
NCC = 2
NSC = 16
NW = NCC * NSC
QBLK = 256
QPW = QBLK // NW
GCAP = 224
GHALF = GCAP // 2
NV_SG = 16
SVCAP = 1024
OUTW = 104

NEG = float("-inf")


def _mm_body(q_ref, c_ref, s_ref, mt_ref):
    j = pl.program_id(1)
    nj = pl.num_programs(1)
    del nj
    s = lax.dot_general(q_ref[...], c_ref[...], (((1,), (1,)), ((), ())),
                        preferred_element_type=jnp.float32)
    s = jnp.where(
        lax.broadcasted_iota(jnp.int32, (QB, CB), 1) + j * CB >= C_REAL,
        NEG, s)
    s3 = s.reshape(QB, CB // CHUNK, CHUNK)
    s_ref[...] = s3
    cm = jnp.max(s3, axis=2)
    mt_ref[...] = cm.T


def _stage1(queries, cand_pad):
    Q, D = queries.shape
    return pl.pallas_call(
        _mm_body,
        grid=(Q // QB, C_PAD // CB),
        in_specs=[
            pl.BlockSpec((QB, D), lambda i, j: (i, 0)),
            pl.BlockSpec((CB, D), lambda i, j: (j, 0)),
        ],
        out_specs=[
            pl.BlockSpec((QB, CB // CHUNK, CHUNK), lambda i, j: (i, j, 0)),
            pl.BlockSpec((CB // CHUNK, QB), lambda i, j: (j, i)),
        ],
        out_shape=[
            jax.ShapeDtypeStruct((Q, NCH, CHUNK), jnp.float32),
            jax.ShapeDtypeStruct((NCH, Q), jnp.float32),
        ],
    )(queries, cand_pad)


def _sort_kv(k, v):
    return plsc.sort_key_val(k, v, descending=True)


def _vsort_desc(x):
    k, _ = _sort_kv(x, x)
    return k


def _rev_run(b):
    return [lax.rev(x, (0,)) for x in reversed(b)]


def _bitonic_finish_k(v):
    n = len(v)
    d = n // 2
    while d >= 1:
        nv = list(v)
        for base in range(0, n, 2 * d):
            for i in range(base, base + d):
                nv[i] = jnp.maximum(v[i], v[i + d])
                nv[i + d] = jnp.minimum(v[i], v[i + d])
        v = nv
        d //= 2
    return [_vsort_desc(x) for x in v]


def _merge_desc_k(a, b):
    return _bitonic_finish_k(a + _rev_run(b))


def _merge_desc_k_top(a, b):
    m = len(a)
    b2 = _rev_run(b)
    v = [jnp.maximum(a[i], b2[i]) for i in range(m)]
    if m == 1:
        return [_vsort_desc(v[0])]
    return _bitonic_finish_k(v)


def _bitonic_finish_kv(ks, vs):
    n = len(ks)
    d = n // 2
    while d >= 1:
        nk, nv = list(ks), list(vs)
        for base in range(0, n, 2 * d):
            for i in range(base, base + d):
                c = ks[i] >= ks[i + d]
                nk[i] = jnp.where(c, ks[i], ks[i + d])
                nv[i] = jnp.where(c, vs[i], vs[i + d])
                nk[i + d] = jnp.where(c, ks[i + d], ks[i])
                nv[i + d] = jnp.where(c, vs[i + d], vs[i])
        ks, vs = nk, nv
        d //= 2
    out = [_sort_kv(k, v) for k, v in zip(ks, vs)]
    return [k for k, _ in out], [v for _, v in out]


def _merge_desc_kv(ka, va, kb, vb):
    return _bitonic_finish_kv(ka + _rev_run(kb), va + _rev_run(vb))


def _merge_desc_kv_top(ka, va, kb, vb):
    m = len(ka)
    kb2, vb2 = _rev_run(kb), _rev_run(vb)
    ks, vs = [], []
    for i in range(m):
        c = ka[i] >= kb2[i]
        ks.append(jnp.where(c, ka[i], kb2[i]))
        vs.append(jnp.where(c, va[i], vb2[i]))
    if m == 1:
        k, v = _sort_kv(ks[0], vs[0])
        return [k], [v]
    return _bitonic_finish_kv(ks, vs)


def _tie_fixup(ov_v, oi_v, phases=4):
    iota = lax.iota(jnp.int32, 16)
    for p in range(phases):
        parity = p % 2
        new_v = []
        for i in range(8):
            pos = 16 * i + iota
            step = jnp.where(pos % 2 == parity, 1, -1)
            partner = jnp.clip(pos + step, 0, 127)
            k = ov_v[pl.ds(16 * i, 16)]
            v = oi_v[pl.ds(16 * i, 16)]
            kp = plsc.load_gather(ov_v, [partner])
            vp = plsc.load_gather(oi_v, [partner])
            eq = k == kp
            lead = partner > pos
            nv = jnp.where(lead, jnp.minimum(v, vp), jnp.maximum(v, vp))
            new_v.append(jnp.where(eq, nv, v))
        for i in range(8):
            oi_v[pl.ds(16 * i, 16)] = new_v[i]


def _topk_ladder_k(vregs, keep):
    runs = [[_vsort_desc(x)] for x in vregs]
    while len(runs) > 1:
        nxt = []
        for i in range(0, len(runs) - 1, 2):
            a, b = runs[i], runs[i + 1]
            if len(a) < keep:
                nxt.append(_merge_desc_k(a, b))
            else:
                nxt.append(_merge_desc_k_top(a, b))
        if len(runs) % 2:
            nxt.append(runs[-1])
        runs = nxt
    return runs[0]


def _sort8_kv(ks, vs):
    runs = []
    for k, v in zip(ks, vs):
        k2, v2 = _sort_kv(k, v)
        runs.append(([k2], [v2]))
    while len(runs) > 1:
        nxt = []
        for i in range(0, len(runs), 2):
            ka, va = runs[i]
            kb, vb = runs[i + 1]
            nxt.append(_merge_desc_kv(ka, va, kb, vb))
        runs = nxt
    return runs[0]


def _sc_body(m_hbm, rows_hbm, outv_hbm, outi_hbm,
             m_v0, m_v1, tau_v, gid_v0, gid_v1, ga0, gb0, ga1, gb1,
             rows_v0, rows_v1, sv_s, sv_i, ov_v, oi_v,
             semm0, semm1, semg0, semg1):
    wid = lax.axis_index("s") * NCC + lax.axis_index("c")
    base = wid * QPW
    iota = lax.iota(jnp.int32, 16)

    def start_mload(q, m_v, semm):
        qq = jnp.minimum(q, QBLK - 1)
        pltpu.async_copy(m_hbm.at[pl.ds(qq * NCHP, NCHP)], m_v, semm)

    def wait_mload(m_v, semm):
        pltpu.make_async_copy(m_hbm.at[pl.ds(0, NCHP)], m_v, semm).wait()

    def start_gather(ga, gb, rows_v, semg):
        pltpu.async_copy(rows_hbm.at[ga], rows_v.at[pl.ds(0, GHALF)], semg)
        pltpu.async_copy(rows_hbm.at[gb], rows_v.at[pl.ds(GHALF, GHALF)],
                         semg)

    def wait_gather(ga, gb, rows_v, semg):
        pltpu.make_async_copy(rows_hbm.at[ga],
                              rows_v.at[pl.ds(0, GHALF)], semg).wait()
        pltpu.make_async_copy(rows_hbm.at[gb],
                              rows_v.at[pl.ds(GHALF, GHALF)], semg).wait()

    def compute_phase(q, m_v, gid_v, ga, gb):
        mv = [m_v[pl.ds(16 * i, 16)] for i in range(NV_M)]
        sg = [jnp.maximum(jnp.maximum(mv[4 * i], mv[4 * i + 1]),
                          jnp.maximum(mv[4 * i + 2], mv[4 * i + 3]))
              for i in range(NV_M // 4)]
        negv = jnp.full((16,), NEG, jnp.float32)
        sg += [negv] * (NV_SG - len(sg))
        run = _topk_ladder_k(sg, 8)
        tau_v[...] = run[6]
        tau = plsc.load_gather(tau_v, [jnp.full((16,), 3, jnp.int32)])

        for i in range(GCAP // 16):
            gid_v[pl.ds(16 * i, 16)] = q * NCH + 16 * i + iota

        def comp_body(i, off):
            mk = m_v[pl.ds(16 * i, 16)]
            msk = mk >= tau
            rowid = q * NCH + 16 * i + iota
            plsc.store_compressed(gid_v.at[pl.ds(off, 16)], rowid, mask=msk)
            cnt = jnp.sum(msk.astype(jnp.int32))
            return jnp.minimum(off + cnt, GCAP - 16)

        nch = lax.fori_loop(0, NV_M, comp_body, jnp.int32(0), unroll=True)
        for i in range(GHALF // 16):
            ga[pl.ds(16 * i, 16)] = gid_v[pl.ds(16 * i, 16)]
            gb[pl.ds(16 * i, 16)] = gid_v[pl.ds(GHALF + 16 * i, 16)]
        return tau, nch

    def finish_phase(q, tau, nch, gid_v, rows_v):
        def filt_body(g, off):
            rid = plsc.load_gather(gid_v, [jnp.full((16,), 0, jnp.int32) + g])
            cbase = (rid - q * NCH) * CHUNK
            for ti in range(CHUNK // 16):
                s = rows_v[g, pl.ds(16 * ti, 16)]
                msk = s >= tau
                plsc.store_compressed(sv_s.at[pl.ds(off, 16)], s, mask=msk)
                idxv = cbase + 16 * ti + iota
                plsc.store_compressed(sv_i.at[pl.ds(off, 16)], idxv, mask=msk)
                off = jnp.minimum(off + jnp.sum(msk.astype(jnp.int32)),
                                  SVCAP - 16)
            return off

        svcnt = lax.fori_loop(0, nch, filt_body, jnp.int32(0))

        def load_group(g):
            ks, vs = [], []
            for ti in range(8):
                pos = 128 * g + 16 * ti + iota
                k = sv_s[pl.ds(128 * g + 16 * ti, 16)]
                ks.append(jnp.where(pos < svcnt, k, NEG))
                vs.append(sv_i[pl.ds(128 * g + 16 * ti, 16)])
            return ks, vs

        ks0, vs0 = load_group(jnp.int32(0))
        rk, rv = _sort8_kv(ks0, vs0)

        def grp_body(g, carry):
            rk = list(carry[:8])
            rv = list(carry[8:])
            ks, vs = load_group(g)
            gk, gv = _sort8_kv(ks, vs)
            rk, rv = _merge_desc_kv_top(rk, rv, gk, gv)
            return tuple(rk) + tuple(rv)

        ngrp = (jnp.minimum(svcnt, SVCAP) + 127) // 128
        carry = lax.fori_loop(1, ngrp, grp_body, tuple(rk) + tuple(rv))
        rk, rv = list(carry[:8]), list(carry[8:])

        for ti in range(8):
            ov_v[pl.ds(16 * ti, 16)] = rk[ti]
            oi_v[pl.ds(16 * ti, 16)] = rv[ti]
        _tie_fixup(ov_v, oi_v)
        pltpu.sync_copy(ov_v.at[pl.ds(0, OUTW)],
                        outv_hbm.at[pl.ds(q * OUTW, OUTW)])
        pltpu.sync_copy(oi_v.at[pl.ds(0, OUTW)],
                        outi_hbm.at[pl.ds(q * OUTW, OUTW)])

    start_mload(base + 0, m_v0, semm0)
    start_mload(base + 1, m_v1, semm1)

    wait_mload(m_v0, semm0)
    tau_e, nch_e = compute_phase(base + 0, m_v0, gid_v0, ga0, gb0)
    start_gather(ga0, gb0, rows_v0, semg0)
    start_mload(base + 2, m_v0, semm0)

    wait_mload(m_v1, semm1)
    tau_o, nch_o = compute_phase(base + 1, m_v1, gid_v1, ga1, gb1)
    start_gather(ga1, gb1, rows_v1, semg1)
    start_mload(base + 3, m_v1, semm1)

    wait_gather(ga0, gb0, rows_v0, semg0)
    finish_phase(base + 0, tau_e, nch_e, gid_v0, rows_v0)

    def pair_body(p, carry):
        tau_o, nch_o = carry
        a = base + 2 * p
        b = a + 1

        wait_mload(m_v0, semm0)
        tau_e, nch_e = compute_phase(a, m_v0, gid_v0, ga0, gb0)
        start_gather(ga0, gb0, rows_v0, semg0)
        start_mload(a + 2, m_v0, semm0)

        wait_gather(ga1, gb1, rows_v1, semg1)
        finish_phase(b - 2, tau_o, nch_o, gid_v1, rows_v1)

        wait_mload(m_v1, semm1)
        tau_o, nch_o = compute_phase(b, m_v1, gid_v1, ga1, gb1)
        start_gather(ga1, gb1, rows_v1, semg1)
        start_mload(b + 2, m_v1, semm1)

        wait_gather(ga0, gb0, rows_v0, semg0)
        finish_phase(a, tau_e, nch_e, gid_v0, rows_v0)
        return (tau_o, nch_o)

    tau_o, nch_o = lax.fori_loop(1, QPW // 2, pair_body, (tau_o, nch_o))

    wait_gather(ga1, gb1, rows_v1, semg1)
    finish_phase(base + QPW - 1, tau_o, nch_o, gid_v1, rows_v1)


def _stage2(m_flat, rows, Q):
    mesh = plsc.VectorSubcoreMesh(core_axis_name="c", subcore_axis_name="s",
                                  num_cores=NCC, num_subcores=NSC)
    f = functools.partial(
        pl.kernel,
        out_type=[
            jax.ShapeDtypeStruct((Q * OUTW,), jnp.float32),
            jax.ShapeDtypeStruct((Q * OUTW,), jnp.int32),
        ],
        mesh=mesh,
        compiler_params=pltpu.CompilerParams(needs_layout_passes=False),
        scratch_types=[
            pltpu.VMEM((NCHP,), jnp.float32),
            pltpu.VMEM((NCHP,), jnp.float32),
            pltpu.VMEM((16,), jnp.float32),
            pltpu.VMEM((GCAP,), jnp.int32),
            pltpu.VMEM((GCAP,), jnp.int32),
            pltpu.VMEM((GHALF,), jnp.int32),
            pltpu.VMEM((GHALF,), jnp.int32),
            pltpu.VMEM((GHALF,), jnp.int32),
            pltpu.VMEM((GHALF,), jnp.int32),
            pltpu.VMEM((GCAP, CHUNK), jnp.float32),
            pltpu.VMEM((GCAP, CHUNK), jnp.float32),
            pltpu.VMEM((SVCAP,), jnp.float32),
            pltpu.VMEM((SVCAP,), jnp.int32),
            pltpu.VMEM((128,), jnp.float32),
            pltpu.VMEM((128,), jnp.int32),
            pltpu.SemaphoreType.DMA,
            pltpu.SemaphoreType.DMA,
            pltpu.SemaphoreType.DMA,
            pltpu.SemaphoreType.DMA,
        ],
    )(_sc_body)
    return f(m_flat, rows)


def kernel(queries, candidates):
    Q, D = queries.shape
    C = candidates.shape[0]
    cand_pad = jnp.pad(candidates, ((0, C_PAD - C), (0, 0)))
    outs = []
    for qb in range(Q // QBLK):
        qblk = queries[qb * QBLK:(qb + 1) * QBLK]
        scores, mt = _stage1(qblk, cand_pad)
        m = jnp.pad(mt.T, ((0, 0), (0, NCHP - NCH)), constant_values=NEG)
        rows = scores.reshape(QBLK * NCH, CHUNK)
        outv, outi = _stage2(m.reshape(-1), rows, QBLK)
        outs.append((outv.reshape(QBLK, OUTW), outi.reshape(QBLK, OUTW)))
    top_scores = jnp.concatenate([o[0] for o in outs])[:, :K_TOP_CONST]
    top_idx = jnp.concatenate([o[1] for o in outs])[:, :K_TOP_CONST]
    return (top_scores, top_idx)


# speedup vs baseline: 6.7196x; 1.0107x over previous
import functools

import jax
import jax.numpy as jnp
from jax import lax
from jax.experimental import pallas as pl
from jax.experimental.pallas import tpu as pltpu
from jax.experimental.pallas import tpu_sc as plsc

K_TOP_CONST = 100

QB = 256
CB = 1024
C_REAL = 100000
C_PAD = 100352
CHUNK = 128
NCH = C_PAD // CHUNK
NCHP = 896
NV_M = NCHP // 16
NCC = 2
NSC = 16
NW = NCC * NSC
QBLK = 256
QPW = QBLK // NW
GCAP = 224
GHALF = GCAP // 2
NV_SG = 16
SVCAP = 1024
OUTW = 104

NEG = float("-inf")


def _mm_body(q_ref, c_ref, s_ref, mt_ref):
    j = pl.program_id(1)
    nj = pl.num_programs(1)
    del nj
    s = lax.dot_general(q_ref[...], c_ref[...], (((1,), (1,)), ((), ())),
                        preferred_element_type=jnp.float32)
    s = jnp.where(
        lax.broadcasted_iota(jnp.int32, (QB, CB), 1) + j * CB >= C_REAL,
        NEG, s)
    s3 = s.reshape(QB, CB // CHUNK, CHUNK)
    s_ref[...] = s3
    cm = jnp.max(s3, axis=2)
    mt_ref[...] = cm.T


def _stage1(queries, cand_pad):
    Q, D = queries.shape
    return pl.pallas_call(
        _mm_body,
        grid=(Q // QB, C_PAD // CB),
        in_specs=[
            pl.BlockSpec((QB, D), lambda i, j: (i, 0)),
            pl.BlockSpec((CB, D), lambda i, j: (j, 0)),
        ],
        out_specs=[
            pl.BlockSpec((QB, CB // CHUNK, CHUNK), lambda i, j: (i, j, 0)),
            pl.BlockSpec((CB // CHUNK, QB), lambda i, j: (j, i)),
        ],
        out_shape=[
            jax.ShapeDtypeStruct((Q, NCH, CHUNK), jnp.float32),
            jax.ShapeDtypeStruct((NCH, Q), jnp.float32),
        ],
    )(queries, cand_pad)


def _sort_kv(k, v):
    return plsc.sort_key_val(k, v, descending=True)


def _vsort_desc(x):
    k, _ = _sort_kv(x, x)
    return k


def _rev_run(b):
    return [lax.rev(x, (0,)) for x in reversed(b)]


def _bitonic_finish_k(v):
    n = len(v)
    d = n // 2
    while d >= 1:
        nv = list(v)
        for base in range(0, n, 2 * d):
            for i in range(base, base + d):
                nv[i] = jnp.maximum(v[i], v[i + d])
                nv[i + d] = jnp.minimum(v[i], v[i + d])
        v = nv
        d //= 2
    return [_vsort_desc(x) for x in v]


def _merge_desc_k(a, b):
    return _bitonic_finish_k(a + _rev_run(b))


def _merge_desc_k_top(a, b):
    m = len(a)
    b2 = _rev_run(b)
    v = [jnp.maximum(a[i], b2[i]) for i in range(m)]
    if m == 1:
        return [_vsort_desc(v[0])]
    return _bitonic_finish_k(v)


def _bitonic_finish_kv(ks, vs):
    n = len(ks)
    d = n // 2
    while d >= 1:
        nk, nv = list(ks), list(vs)
        for base in range(0, n, 2 * d):
            for i in range(base, base + d):
                c = ks[i] >= ks[i + d]
                nk[i] = jnp.where(c, ks[i], ks[i + d])
                nv[i] = jnp.where(c, vs[i], vs[i + d])
                nk[i + d] = jnp.where(c, ks[i + d], ks[i])
                nv[i + d] = jnp.where(c, vs[i + d], vs[i])
        ks, vs = nk, nv
        d //= 2
    out = [_sort_kv(k, v) for k, v in zip(ks, vs)]
    return [k for k, _ in out], [v for _, v in out]


def _merge_desc_kv(ka, va, kb, vb):
    return _bitonic_finish_kv(ka + _rev_run(kb), va + _rev_run(vb))


def _merge_desc_kv_top(ka, va, kb, vb):
    m = len(ka)
    kb2, vb2 = _rev_run(kb), _rev_run(vb)
    ks, vs = [], []
    for i in range(m):
        c = ka[i] >= kb2[i]
        ks.append(jnp.where(c, ka[i], kb2[i]))
        vs.append(jnp.where(c, va[i], vb2[i]))
    if m == 1:
        k, v = _sort_kv(ks[0], vs[0])
        return [k], [v]
    return _bitonic_finish_kv(ks, vs)


def _tie_fixup(ov_v, oi_v, phases=4):
    iota = lax.iota(jnp.int32, 16)
    for p in range(phases):
        parity = p % 2
        new_v = []
        for i in range(8):
            pos = 16 * i + iota
            step = jnp.where(pos % 2 == parity, 1, -1)
            partner = jnp.clip(pos + step, 0, 127)
            k = ov_v[pl.ds(16 * i, 16)]
            v = oi_v[pl.ds(16 * i, 16)]
            kp = plsc.load_gather(ov_v, [partner])
            vp = plsc.load_gather(oi_v, [partner])
            eq = k == kp
            lead = partner > pos
            nv = jnp.where(lead, jnp.minimum(v, vp), jnp.maximum(v, vp))
            new_v.append(jnp.where(eq, nv, v))
        for i in range(8):
            oi_v[pl.ds(16 * i, 16)] = new_v[i]


def _topk_ladder_k(vregs, keep):
    runs = [[_vsort_desc(x)] for x in vregs]
    while len(runs) > 1:
        nxt = []
        for i in range(0, len(runs) - 1, 2):
            a, b = runs[i], runs[i + 1]
            if len(a) < keep:
                nxt.append(_merge_desc_k(a, b))
            else:
                nxt.append(_merge_desc_k_top(a, b))
        if len(runs) % 2:
            nxt.append(runs[-1])
        runs = nxt
    return runs[0]


def _sort8_kv(ks, vs):
    runs = []
    for k, v in zip(ks, vs):
        k2, v2 = _sort_kv(k, v)
        runs.append(([k2], [v2]))
    while len(runs) > 1:
        nxt = []
        for i in range(0, len(runs), 2):
            ka, va = runs[i]
            kb, vb = runs[i + 1]
            nxt.append(_merge_desc_kv(ka, va, kb, vb))
        runs = nxt
    return runs[0]


def _sc_body(m_hbm, rows_hbm, outv_hbm, outi_hbm,
             m_v0, m_v1, tau_v, gid_v0, gid_v1, ga0, gb0, ga1, gb1,
             rows_v0, rows_v1, sv_s, sv_i, ov_v, oi_v,
             semm0, semm1, semg0, semg1):
    wid = lax.axis_index("s") * NCC + lax.axis_index("c")
    base = wid * QPW
    iota = lax.iota(jnp.int32, 16)

    def start_mload(q, m_v, semm):
        qq = jnp.minimum(q, QBLK - 1)
        pltpu.async_copy(m_hbm.at[pl.ds(qq * NCHP, NCHP)], m_v, semm)

    def wait_mload(m_v, semm):
        pltpu.make_async_copy(m_hbm.at[pl.ds(0, NCHP)], m_v, semm).wait()

    def start_gather(ga, gb, rows_v, semg):
        pltpu.async_copy(rows_hbm.at[ga], rows_v.at[pl.ds(0, GHALF)], semg)
        pltpu.async_copy(rows_hbm.at[gb], rows_v.at[pl.ds(GHALF, GHALF)],
                         semg)

    def wait_gather(ga, gb, rows_v, semg):
        pltpu.make_async_copy(rows_hbm.at[ga],
                              rows_v.at[pl.ds(0, GHALF)], semg).wait()
        pltpu.make_async_copy(rows_hbm.at[gb],
                              rows_v.at[pl.ds(GHALF, GHALF)], semg).wait()

    def compute_phase(q, m_v, gid_v, ga, gb):
        mv = [m_v[pl.ds(16 * i, 16)] for i in range(NV_M)]
        sg = [jnp.maximum(jnp.maximum(mv[4 * i], mv[4 * i + 1]),
                          jnp.maximum(mv[4 * i + 2], mv[4 * i + 3]))
              for i in range(NV_M // 4)]
        negv = jnp.full((16,), NEG, jnp.float32)
        sg += [negv] * (NV_SG - len(sg))
        run = _topk_ladder_k(sg, 8)
        tau_v[...] = run[6]
        tau = plsc.load_gather(tau_v, [jnp.full((16,), 3, jnp.int32)])

        for i in range(GCAP // 16):
            gid_v[pl.ds(16 * i, 16)] = q * NCH + 16 * i + iota

        def comp_body(i, off):
            mk = m_v[pl.ds(16 * i, 16)]
            msk = mk >= tau
            rowid = q * NCH + 16 * i + iota
            plsc.store_compressed(gid_v.at[pl.ds(off, 16)], rowid, mask=msk)
            cnt = jnp.sum(msk.astype(jnp.int32))
            return jnp.minimum(off + cnt, GCAP - 16)

        nch = lax.fori_loop(0, NV_M, comp_body, jnp.int32(0), unroll=True)
        for i in range(GHALF // 16):
            ga[pl.ds(16 * i, 16)] = gid_v[pl.ds(16 * i, 16)]
            gb[pl.ds(16 * i, 16)] = gid_v[pl.ds(GHALF + 16 * i, 16)]
        return tau, nch

    def finish_phase(q, tau, nch, gid_v, rows_v):
        def filt_body(g, off):
            rid = plsc.load_gather(gid_v, [jnp.full((16,), 0, jnp.int32) + g])
            cbase = (rid - q * NCH) * CHUNK
            for ti in range(CHUNK // 16):
                s = rows_v[g, pl.ds(16 * ti, 16)]
                msk = s >= tau
                plsc.store_compressed(sv_s.at[pl.ds(off, 16)], s, mask=msk)
                idxv = cbase + 16 * ti + iota
                plsc.store_compressed(sv_i.at[pl.ds(off, 16)], idxv, mask=msk)
                off = jnp.minimum(off + jnp.sum(msk.astype(jnp.int32)),
                                  SVCAP - 16)
            return off

        svcnt = lax.fori_loop(0, nch, filt_body, jnp.int32(0))

        def load_group(g):
            ks, vs = [], []
            for ti in range(8):
                pos = 128 * g + 16 * ti + iota
                k = sv_s[pl.ds(128 * g + 16 * ti, 16)]
                ks.append(jnp.where(pos < svcnt, k, NEG))
                vs.append(sv_i[pl.ds(128 * g + 16 * ti, 16)])
            return ks, vs

        ks0, vs0 = load_group(jnp.int32(0))
        rk, rv = _sort8_kv(ks0, vs0)

        def grp_body(g, carry):
            rk = list(carry[:8])
            rv = list(carry[8:])
            ks, vs = load_group(g)
            gk, gv = _sort8_kv(ks, vs)
            rk, rv = _merge_desc_kv_top(rk, rv, gk, gv)
            return tuple(rk) + tuple(rv)

        ngrp = (jnp.minimum(svcnt, SVCAP) + 127) // 128
        carry = lax.fori_loop(1, ngrp, grp_body, tuple(rk) + tuple(rv))
        rk, rv = list(carry[:8]), list(carry[8:])

        for ti in range(8):
            ov_v[pl.ds(16 * ti, 16)] = rk[ti]
            oi_v[pl.ds(16 * ti, 16)] = rv[ti]
        _tie_fixup(ov_v, oi_v)
        pltpu.sync_copy(ov_v.at[pl.ds(0, OUTW)],
                        outv_hbm.at[pl.ds(q * OUTW, OUTW)])
        pltpu.sync_copy(oi_v.at[pl.ds(0, OUTW)],
                        outi_hbm.at[pl.ds(q * OUTW, OUTW)])

    start_mload(base + 0, m_v0, semm0)
    start_mload(base + 1, m_v1, semm1)

    wait_mload(m_v0, semm0)
    tau_e, nch_e = compute_phase(base + 0, m_v0, gid_v0, ga0, gb0)
    start_gather(ga0, gb0, rows_v0, semg0)
    start_mload(base + 2, m_v0, semm0)

    wait_mload(m_v1, semm1)
    tau_o, nch_o = compute_phase(base + 1, m_v1, gid_v1, ga1, gb1)
    start_gather(ga1, gb1, rows_v1, semg1)
    start_mload(base + 3, m_v1, semm1)

    wait_gather(ga0, gb0, rows_v0, semg0)
    finish_phase(base + 0, tau_e, nch_e, gid_v0, rows_v0)

    def pair_body(p, carry):
        tau_o, nch_o = carry
        a = base + 2 * p
        b = a + 1

        wait_mload(m_v0, semm0)
        tau_e, nch_e = compute_phase(a, m_v0, gid_v0, ga0, gb0)
        start_gather(ga0, gb0, rows_v0, semg0)
        start_mload(a + 2, m_v0, semm0)

        wait_gather(ga1, gb1, rows_v1, semg1)
        finish_phase(b - 2, tau_o, nch_o, gid_v1, rows_v1)

        wait_mload(m_v1, semm1)
        tau_o, nch_o = compute_phase(b, m_v1, gid_v1, ga1, gb1)
        start_gather(ga1, gb1, rows_v1, semg1)
        start_mload(b + 2, m_v1, semm1)

        wait_gather(ga0, gb0, rows_v0, semg0)
        finish_phase(a, tau_e, nch_e, gid_v0, rows_v0)
        return (tau_o, nch_o)

    tau_o, nch_o = lax.fori_loop(1, QPW // 2, pair_body, (tau_o, nch_o))

    wait_gather(ga1, gb1, rows_v1, semg1)
    finish_phase(base + QPW - 1, tau_o, nch_o, gid_v1, rows_v1)


def _stage2(m_flat, rows, Q):
    mesh = plsc.VectorSubcoreMesh(core_axis_name="c", subcore_axis_name="s",
                                  num_cores=NCC, num_subcores=NSC)
    f = functools.partial(
        pl.kernel,
        out_type=[
            jax.ShapeDtypeStruct((Q * OUTW,), jnp.float32),
            jax.ShapeDtypeStruct((Q * OUTW,), jnp.int32),
        ],
        mesh=mesh,
        compiler_params=pltpu.CompilerParams(needs_layout_passes=False),
        scratch_types=[
            pltpu.VMEM((NCHP,), jnp.float32),
            pltpu.VMEM((NCHP,), jnp.float32),
            pltpu.VMEM((16,), jnp.float32),
            pltpu.VMEM((GCAP,), jnp.int32),
            pltpu.VMEM((GCAP,), jnp.int32),
            pltpu.VMEM((GHALF,), jnp.int32),
            pltpu.VMEM((GHALF,), jnp.int32),
            pltpu.VMEM((GHALF,), jnp.int32),
            pltpu.VMEM((GHALF,), jnp.int32),
            pltpu.VMEM((GCAP, CHUNK), jnp.float32),
            pltpu.VMEM((GCAP, CHUNK), jnp.float32),
            pltpu.VMEM((SVCAP,), jnp.float32),
            pltpu.VMEM((SVCAP,), jnp.int32),
            pltpu.VMEM((128,), jnp.float32),
            pltpu.VMEM((128,), jnp.int32),
            pltpu.SemaphoreType.DMA,
            pltpu.SemaphoreType.DMA,
            pltpu.SemaphoreType.DMA,
            pltpu.SemaphoreType.DMA,
        ],
    )(_sc_body)
    return f(m_flat, rows)


def kernel(queries, candidates):
    Q, D = queries.shape
    C = candidates.shape[0]
    cand_pad = jnp.pad(candidates, ((0, C_PAD - C), (0, 0)))
    outs = []
    dep = jnp.float32(0.0)
    for qb in range(Q // QBLK):
        qblk = queries[qb * QBLK:(qb + 1) * QBLK]
        scores, mt = _stage1(qblk, cand_pad)
        m = jnp.pad(mt.T, ((0, 0), (0, NCHP - NCH)), constant_values=NEG)
        rows = scores.reshape(QBLK * NCH, CHUNK)
        outv, outi = _stage2(m.reshape(-1) + 0.0 * dep, rows, QBLK)
        dep = outv[0]
        outs.append((outv.reshape(QBLK, OUTW), outi.reshape(QBLK, OUTW)))
    top_scores = jnp.concatenate([o[0] for o in outs])[:, :K_TOP_CONST]
    top_idx = jnp.concatenate([o[1] for o in outs])[:, :K_TOP_CONST]
    return (top_scores, top_idx)
